# baseline (speedup 1.0000x reference)
import functools

import jax
import jax.numpy as jnp
from jax.experimental import pallas as pl
from jax.experimental.pallas import tpu as pltpu

N_ROWS = 5000
D_IN = 12544
D_HID = 1024
BR = 200
NR = N_ROWS // BR
WSLAB = 896
NW = D_IN // WSLAB
NSTEPS = NW + NR
C1 = 4
C4 = 12
CH = 16


def _boxhead_body(x_ref, w1_ref, b1_ref, w2_ref, b2_ref, wh_ref, bh_ref,
                  out_ref, w1b_ref, acc0_ref):
    j = pl.program_id(0)

    def _epilogue(h1):
        h2 = jnp.maximum(
            jnp.dot(h1.astype(jnp.bfloat16), w2_ref[...],
                    preferred_element_type=jnp.float32)
            + b2_ref[...], 0.0)
        out_ref[...] = (jnp.dot(h2.astype(jnp.bfloat16), wh_ref[...],
                                preferred_element_type=jnp.float32)
                        + bh_ref[...])

    @pl.when(j < NW)
    def _cast_w1():
        wslab = w1_ref[...].astype(jnp.bfloat16)
        w1b_ref[pl.ds(j * WSLAB, WSLAB), :] = wslab
        xb0 = x_ref[:, pl.ds(j * WSLAB, WSLAB)].astype(jnp.bfloat16)
        part = jnp.dot(xb0, wslab, preferred_element_type=jnp.float32)

        @pl.when(j == 0)
        def _():
            acc0_ref[...] = part.astype(jnp.bfloat16)

        @pl.when(j > 0)
        def _():
            acc0_ref[...] = (acc0_ref[...].astype(jnp.float32)
                             + part).astype(jnp.bfloat16)

    @pl.when(j == NW)
    def _stripe0():
        _epilogue(jnp.maximum(
            acc0_ref[...].astype(jnp.float32) + b1_ref[...], 0.0))

    @pl.when(j > NW)
    def _stripe():
        xb = x_ref[...].astype(jnp.bfloat16)
        h1 = jnp.maximum(
            jnp.dot(xb, w1b_ref[...], preferred_element_type=jnp.float32)
            + b1_ref[...], 0.0)
        _epilogue(h1)


def _clamp(lo, v, hi):
    return jnp.minimum(jnp.maximum(v, lo), hi)


@functools.partial(jax.jit, static_argnames=())
def kernel(feature_vectors, W1, b1, W2, b2, W3, b3, W4, b4):
    W2b = W2.astype(jnp.bfloat16)
    WHb = jnp.concatenate([W3, W4], axis=1).astype(jnp.bfloat16)
    bh = jnp.concatenate([b3, b4]).reshape(1, CH)
    out = pl.pallas_call(
        _boxhead_body,
        grid=(NSTEPS,),
        in_specs=[
            pl.BlockSpec((BR, D_IN),
                         lambda j: (_clamp(0, j - NW, NR - 1), 0)),
            pl.BlockSpec((WSLAB, D_HID),
                         lambda j: (_clamp(0, j, NW - 1), 0)),
            pl.BlockSpec((1, D_HID), lambda j: (0, 0)),
            pl.BlockSpec((D_HID, D_HID), lambda j: (0, 0)),
            pl.BlockSpec((1, D_HID), lambda j: (0, 0)),
            pl.BlockSpec((D_HID, CH), lambda j: (0, 0)),
            pl.BlockSpec((1, CH), lambda j: (0, 0)),
        ],
        out_specs=pl.BlockSpec((BR, CH), lambda j: (_clamp(0, j - NW, NR - 1), 0)),
        out_shape=jax.ShapeDtypeStruct((N_ROWS, CH), jnp.float32),
        scratch_shapes=[
            pltpu.VMEM((D_IN, D_HID), jnp.bfloat16),
            pltpu.VMEM((BR, D_HID), jnp.bfloat16),
        ],
        compiler_params=pltpu.CompilerParams(
            dimension_semantics=("arbitrary",),
        ),
    )(feature_vectors, W1, b1.reshape(1, -1), W2b, b2.reshape(1, -1),
      WHb, bh)
    return (out[:, :C1], out[:, C1:])


# speedup vs baseline: 1.0116x; 1.0116x over previous
import functools

import jax
import jax.numpy as jnp
from jax.experimental import pallas as pl
from jax.experimental.pallas import tpu as pltpu

N_ROWS = 5000
D_IN = 12544
D_HID = 1024
BR = 200
NR = N_ROWS // BR
WSLAB = 896
NW = D_IN // WSLAB
NSTEPS = NW + NR
C1 = 4
C4 = 12
CH = 16


def _boxhead_body(x_ref, w1_ref, b1_ref, w2_ref, b2_ref, wh_ref, bh_ref,
                  out_ref, w1b_ref):
    j = pl.program_id(0)

    @pl.when(j < NW)
    def _cast_w1():
        w1b_ref[pl.ds(j * WSLAB, WSLAB), :] = w1_ref[...].astype(jnp.bfloat16)

    @pl.when(j >= NW)
    def _stripe():
        xb = x_ref[...].astype(jnp.bfloat16)
        h1 = jnp.maximum(
            jnp.dot(xb, w1b_ref[...], preferred_element_type=jnp.float32)
            + b1_ref[...], 0.0)
        h2 = jnp.maximum(
            jnp.dot(h1.astype(jnp.bfloat16), w2_ref[...],
                    preferred_element_type=jnp.float32)
            + b2_ref[...], 0.0)
        out_ref[...] = (jnp.dot(h2.astype(jnp.bfloat16), wh_ref[...],
                                preferred_element_type=jnp.float32)
                        + bh_ref[...])


def _clamp(lo, v, hi):
    return jnp.minimum(jnp.maximum(v, lo), hi)


@functools.partial(jax.jit, static_argnames=())
def kernel(feature_vectors, W1, b1, W2, b2, W3, b3, W4, b4):
    W2b = W2.astype(jnp.bfloat16)
    WHb = jnp.concatenate([W3, W4], axis=1).astype(jnp.bfloat16)
    bh = jnp.concatenate([b3, b4]).reshape(1, CH)
    out = pl.pallas_call(
        _boxhead_body,
        grid=(NSTEPS,),
        in_specs=[
            pl.BlockSpec((BR, D_IN),
                         lambda j: (_clamp(0, j - NW, NR - 1), 0)),
            pl.BlockSpec((WSLAB, D_HID),
                         lambda j: (_clamp(0, j, NW - 1), 0)),
            pl.BlockSpec((1, D_HID), lambda j: (0, 0)),
            pl.BlockSpec((D_HID, D_HID), lambda j: (0, 0)),
            pl.BlockSpec((1, D_HID), lambda j: (0, 0)),
            pl.BlockSpec((D_HID, CH), lambda j: (0, 0)),
            pl.BlockSpec((1, CH), lambda j: (0, 0)),
        ],
        out_specs=pl.BlockSpec((BR, CH), lambda j: (_clamp(0, j - NW, NR - 1), 0)),
        out_shape=jax.ShapeDtypeStruct((N_ROWS, CH), jnp.float32),
        scratch_shapes=[
            pltpu.VMEM((D_IN, D_HID), jnp.bfloat16),
        ],
        compiler_params=pltpu.CompilerParams(
            dimension_semantics=("arbitrary",),
        ),
    )(feature_vectors, W1, b1.reshape(1, -1), W2b, b2.reshape(1, -1),
      WHb, bh)
    return (out[:, :C1], out[:, C1:])
